# Initial kernel scaffold; baseline (speedup 1.0000x reference)
import jax
import jax.numpy as jnp
from jax.experimental import pallas as pl


def kernel(x, table):
    raise NotImplementedError("write your pallas kernel here")


# speedup vs baseline: 2.4144x; 2.4144x over previous
import functools
from math import sqrt

import jax
import jax.numpy as jnp
from jax import lax
from jax.experimental import pallas as pl
from jax.experimental.pallas import tpu as pltpu
from jax.experimental.pallas import tpu_sc as plsc

VOCAB = 100000
DIM = 128
SCALE = float(sqrt(DIM))

NC = 2
NS = 16
NW = NC * NS

B_TOTAL = 4096 * 50
BPW = B_TOTAL // NW
CHUNK = 128
NCHUNK = BPW // CHUNK

_mesh = plsc.VectorSubcoreMesh(core_axis_name="c", subcore_axis_name="s")


@functools.partial(
    pl.kernel,
    mesh=_mesh,
    out_type=jax.ShapeDtypeStruct((B_TOTAL, DIM), jnp.float32),
    scratch_types=[
        pltpu.VMEM((NCHUNK, CHUNK), jnp.int32),
        pltpu.VMEM((CHUNK, DIM), jnp.float32),
        pltpu.SemaphoreType.DMA,
    ],
)
def _gather_scale(idx_hbm, table_hbm, out_hbm, idx_v, rows_v, gsem):
    wid = lax.axis_index("s") * NC + lax.axis_index("c")
    base = wid * BPW
    pltpu.sync_copy(idx_hbm.at[wid], idx_v)

    def chunk_body(c, _):
        pltpu.async_copy(table_hbm.at[idx_v.at[c]], rows_v, gsem).wait()

        def scale_row(i, _):
            for jj in range(DIM // 16):
                s = pl.ds(jj * 16, 16)
                rows_v[i, s] = rows_v[i, s] * SCALE
            return 0

        lax.fori_loop(0, CHUNK, scale_row, 0)
        pltpu.sync_copy(rows_v, out_hbm.at[pl.ds(base + c * CHUNK, CHUNK)])
        return 0

    lax.fori_loop(0, NCHUNK, chunk_body, 0)


def kernel(x, table):
    idx = x.reshape(NW, NCHUNK, CHUNK).astype(jnp.int32)
    out = _gather_scale(idx, table)
    return out.reshape(x.shape[0], x.shape[1], DIM)


# speedup vs baseline: 2.9439x; 1.2193x over previous
import functools
from math import sqrt

import jax
import jax.numpy as jnp
from jax import lax
from jax.experimental import pallas as pl
from jax.experimental.pallas import tpu as pltpu
from jax.experimental.pallas import tpu_sc as plsc

VOCAB = 100000
DIM = 128
SCALE = float(sqrt(DIM))

NC = 2
NS = 16
NW = NC * NS

B_TOTAL = 4096 * 50
BPW = B_TOTAL // NW
CHUNK = 128
NCHUNK = BPW // CHUNK

_mesh = plsc.VectorSubcoreMesh(core_axis_name="c", subcore_axis_name="s")


@functools.partial(
    pl.kernel,
    mesh=_mesh,
    out_type=jax.ShapeDtypeStruct((B_TOTAL, DIM), jnp.float32),
    scratch_types=[
        pltpu.VMEM((NCHUNK, CHUNK), jnp.int32),
        pltpu.VMEM((2, CHUNK, DIM), jnp.float32),
        pltpu.VMEM((2, CHUNK, DIM), jnp.float32),
        pltpu.SemaphoreType.DMA,
        pltpu.SemaphoreType.DMA,
    ],
)
def _gather_scale(idx_hbm, table_hbm, out_hbm, idx_v, gbuf, obuf, gsem, osem):
    wid = lax.axis_index("s") * NC + lax.axis_index("c")
    base = wid * BPW
    pltpu.sync_copy(idx_hbm.at[wid], idx_v)

    pltpu.async_copy(table_hbm.at[idx_v.at[0]], gbuf.at[0], gsem)
    pltpu.async_copy(table_hbm.at[idx_v.at[1]], gbuf.at[1], gsem)

    def pair_body(p, _):
        c0 = 2 * p
        for b in range(2):
            c = c0 + b
            pltpu.make_async_copy(
                table_hbm.at[idx_v.at[c]], gbuf.at[b], gsem).wait()

            @pl.when(c >= 2)
            def _wait_ocopy():
                pltpu.make_async_copy(
                    obuf.at[b],
                    out_hbm.at[pl.ds(base + (c - 2) * CHUNK, CHUNK)],
                    osem).wait()

            def scale_row(i, _):
                for jj in range(DIM // 16):
                    s = pl.ds(jj * 16, 16)
                    obuf[b, i, s] = gbuf[b, i, s] * SCALE
                return 0

            lax.fori_loop(0, CHUNK, scale_row, 0)

            @pl.when(c + 2 < NCHUNK)
            def _next_gather():
                pltpu.async_copy(
                    table_hbm.at[idx_v.at[c + 2]], gbuf.at[b], gsem)

            pltpu.async_copy(
                obuf.at[b], out_hbm.at[pl.ds(base + c * CHUNK, CHUNK)], osem)
        return 0

    lax.fori_loop(0, NCHUNK // 2, pair_body, 0)

    for b in range(2):
        c = NCHUNK - 2 + b
        pltpu.make_async_copy(
            obuf.at[b], out_hbm.at[pl.ds(base + c * CHUNK, CHUNK)],
            osem).wait()


def kernel(x, table):
    idx = x.reshape(NW, NCHUNK, CHUNK).astype(jnp.int32)
    out = _gather_scale(idx, table)
    return out.reshape(x.shape[0], x.shape[1], DIM)


# speedup vs baseline: 5.1690x; 1.7558x over previous
import functools
from math import sqrt

import jax
import jax.numpy as jnp
from jax import lax
from jax.experimental import pallas as pl
from jax.experimental.pallas import tpu as pltpu
from jax.experimental.pallas import tpu_sc as plsc

VOCAB = 100000
DIM = 128
SCALE = float(sqrt(DIM))

NC = 2
NS = 16
NW = NC * NS

NBATCH = 4096
SEQ = 50
BPW = NBATCH // NW
GB = 2
CHUNK = GB * SEQ
NCHUNK = BPW // GB

_mesh = plsc.VectorSubcoreMesh(core_axis_name="c", subcore_axis_name="s")


@functools.partial(
    pl.kernel,
    mesh=_mesh,
    out_type=jax.ShapeDtypeStruct((NBATCH, SEQ, DIM), jnp.float32),
    scratch_types=[
        pltpu.VMEM((NCHUNK, CHUNK), jnp.int32),
        pltpu.VMEM((2, CHUNK, DIM), jnp.float32),
        pltpu.VMEM((2, GB, SEQ, DIM), jnp.float32),
        pltpu.SemaphoreType.DMA,
        pltpu.SemaphoreType.DMA,
    ],
)
def _gather_scale(idx_hbm, table_hbm, out_hbm, idx_v, gbuf, obuf, gsem, osem):
    wid = lax.axis_index("s") * NC + lax.axis_index("c")
    base = wid * BPW
    pltpu.sync_copy(idx_hbm.at[wid], idx_v)

    pltpu.async_copy(table_hbm.at[idx_v.at[0]], gbuf.at[0], gsem)
    pltpu.async_copy(table_hbm.at[idx_v.at[1]], gbuf.at[1], gsem)

    def pair_body(p, _):
        c0 = 2 * p
        for b in range(2):
            c = c0 + b
            pltpu.make_async_copy(
                table_hbm.at[idx_v.at[c]], gbuf.at[b], gsem).wait()

            @pl.when(c >= 2)
            def _wait_ocopy():
                pltpu.make_async_copy(
                    obuf.at[b],
                    out_hbm.at[pl.ds(base + (c - 2) * GB, GB)],
                    osem).wait()

            def scale_row(i, _):
                for bb in range(GB):
                    for jj in range(DIM // 16):
                        s = pl.ds(jj * 16, 16)
                        obuf[b, bb, i, s] = gbuf[b, bb * SEQ + i, s] * SCALE
                return 0

            lax.fori_loop(0, SEQ, scale_row, 0)

            @pl.when(c + 2 < NCHUNK)
            def _next_gather():
                pltpu.async_copy(
                    table_hbm.at[idx_v.at[c + 2]], gbuf.at[b], gsem)

            pltpu.async_copy(
                obuf.at[b], out_hbm.at[pl.ds(base + c * GB, GB)], osem)
        return 0

    lax.fori_loop(0, NCHUNK // 2, pair_body, 0)

    for b in range(2):
        c = NCHUNK - 2 + b
        pltpu.make_async_copy(
            obuf.at[b], out_hbm.at[pl.ds(base + c * GB, GB)],
            osem).wait()


def kernel(x, table):
    idx = x.reshape(NW, NCHUNK, CHUNK).astype(jnp.int32)
    return _gather_scale(idx, table)


# speedup vs baseline: 5.1763x; 1.0014x over previous
import functools
from math import sqrt

import jax
import jax.numpy as jnp
from jax import lax
from jax.experimental import pallas as pl
from jax.experimental.pallas import tpu as pltpu
from jax.experimental.pallas import tpu_sc as plsc

VOCAB = 100000
DIM = 128
SCALE = float(sqrt(DIM))

NC = 2
NS = 16
NW = NC * NS

NBATCH = 4096
SEQ = 50
BPW = NBATCH // NW
GB = 2
CHUNK = GB * SEQ
NCHUNK = BPW // GB

_mesh = plsc.VectorSubcoreMesh(core_axis_name="c", subcore_axis_name="s")


@functools.partial(
    pl.kernel,
    mesh=_mesh,
    out_type=jax.ShapeDtypeStruct((NBATCH, SEQ, DIM), jnp.float32),
    compiler_params=pltpu.CompilerParams(use_tc_tiling_on_sc=True),
    scratch_types=[
        pltpu.VMEM((NCHUNK, CHUNK), jnp.int32),
        pltpu.VMEM((2, CHUNK, DIM), jnp.float32),
        pltpu.VMEM((2, GB, SEQ, DIM), jnp.float32),
        pltpu.SemaphoreType.DMA,
        pltpu.SemaphoreType.DMA,
    ],
)
def _gather_scale(idx_hbm, table_hbm, out_hbm, idx_v, gbuf, obuf, gsem, osem):
    wid = lax.axis_index("s") * NC + lax.axis_index("c")
    base = wid * BPW
    pltpu.sync_copy(idx_hbm.at[wid], idx_v)

    pltpu.async_copy(table_hbm.at[idx_v.at[0]], gbuf.at[0], gsem)
    pltpu.async_copy(table_hbm.at[idx_v.at[1]], gbuf.at[1], gsem)

    def pair_body(p, _):
        c0 = 2 * p
        for b in range(2):
            c = c0 + b
            pltpu.make_async_copy(
                table_hbm.at[idx_v.at[c]], gbuf.at[b], gsem).wait()

            @pl.when(c >= 2)
            def _wait_ocopy():
                pltpu.make_async_copy(
                    obuf.at[b],
                    out_hbm.at[pl.ds(base + (c - 2) * GB, GB)],
                    osem).wait()

            def scale_row(i, _):
                for bb in range(GB):
                    for jj in range(DIM // 16):
                        s = pl.ds(jj * 16, 16)
                        obuf[b, bb, i, s] = gbuf[b, bb * SEQ + i, s] * SCALE
                return 0

            lax.fori_loop(0, SEQ, scale_row, 0)

            @pl.when(c + 2 < NCHUNK)
            def _next_gather():
                pltpu.async_copy(
                    table_hbm.at[idx_v.at[c + 2]], gbuf.at[b], gsem)

            pltpu.async_copy(
                obuf.at[b], out_hbm.at[pl.ds(base + c * GB, GB)], osem)
        return 0

    lax.fori_loop(0, NCHUNK // 2, pair_body, 0)

    for b in range(2):
        c = NCHUNK - 2 + b
        pltpu.make_async_copy(
            obuf.at[b], out_hbm.at[pl.ds(base + c * GB, GB)],
            osem).wait()


def kernel(x, table):
    idx = x.reshape(NW, NCHUNK, CHUNK).astype(jnp.int32)
    return _gather_scale(idx, table)
